# Initial kernel scaffold; baseline (speedup 1.0000x reference)
import jax
import jax.numpy as jnp
from jax.experimental import pallas as pl


def kernel(x, edge_index, batch, W0, b0, W1, b1, W2, b2, Wh, bh, Wo, bo):
    raise NotImplementedError("write your pallas kernel here")


# speedup vs baseline: 1.0006x; 1.0006x over previous
import jax
import jax.numpy as jnp
from jax.experimental import pallas as pl

_N = 10000
_G = 64
_T = 20


def _unitary_prop(hr, hi, src, dst, norm, n_nodes):
    def spmv(v):
        msg = jnp.take(v, src, axis=0) * norm[:, None]
        return jax.ops.segment_sum(msg, dst, num_segments=n_nodes)
    acc_r, acc_i = hr, hi
    p_r, p_i = hr, hi
    fact = 1.0
    for t in range(1, _T):
        p_r = spmv(p_r)
        p_i = spmv(p_i)
        fact = fact * t
        c = 1.0 / fact
        m = t % 4
        if m == 0:
            acc_r = acc_r + c * p_r; acc_i = acc_i + c * p_i
        elif m == 1:
            acc_r = acc_r - c * p_i; acc_i = acc_i + c * p_r
        elif m == 2:
            acc_r = acc_r - c * p_r; acc_i = acc_i - c * p_i
        else:
            acc_r = acc_r + c * p_i; acc_i = acc_i - c * p_r
    return acc_r, acc_i


def _head_kernel(pooled_ref, wh_ref, bh_ref, wo_ref, bo_ref, out_ref):
    h = jnp.maximum(pooled_ref[...] @ wh_ref[...] + bh_ref[...][None, :], 0.0)
    out_ref[...] = h @ wo_ref[...] + bo_ref[...][None, :]


def kernel(x, edge_index, batch, W0, b0, W1, b1, W2, b2, Wh, bh, Wo, bo):
    src = edge_index[0]
    dst = edge_index[1]
    deg = jnp.bincount(dst, length=_N).astype(jnp.float32) + 1.0
    norm = 1.0 / jnp.sqrt(jnp.take(deg, src) * jnp.take(deg, dst))
    hr = x @ W0 + b0
    hi = jnp.zeros_like(hr)
    hr, hi = _unitary_prop(hr, hi, src, dst, norm, _N)
    for W, b in ((W1, b1), (W2, b2)):
        hr, hi = hr @ W + b, hi @ W
        hr, hi = _unitary_prop(hr, hi, src, dst, norm, _N)
    pooled = jax.ops.segment_sum(hr, batch, num_segments=_G)
    counts = jnp.clip(jnp.bincount(batch, length=_G).astype(jnp.float32), 1.0, None)
    pooled = pooled / counts[:, None]
    out = pl.pallas_call(
        _head_kernel,
        out_shape=jax.ShapeDtypeStruct((_G, Wo.shape[1]), jnp.float32),
    )(pooled, Wh, bh, Wo, bo)
    return out


# speedup vs baseline: 9.8763x; 9.8704x over previous
import functools

import jax
import jax.numpy as jnp
from jax import lax
from jax.experimental import pallas as pl
from jax.experimental.pallas import tpu as pltpu
from jax.experimental.pallas import tpu_sc as plsc

_N = 10000
_E = 160000
_D = 128
_H = 128
_OUT = 40
_G = 64
_T = 20

_NTEC = 16
_NP = 10240
_RPT = _NP // _NTEC
_RCH = 128
_NRC = _RPT // _RCH
_ECH = 128
_EPT = _E // _NTEC
_NEC = -(-_EPT // _ECH)
_NEC = ((_NEC + 1) // 2) * 2
_TRASH = 8


def _sc_taylor(u0, dsti, srci, rdeg16, coefs):
    mesh = plsc.VectorSubcoreMesh(core_axis_name="c", subcore_axis_name="s")

    @functools.partial(
        pl.kernel,
        mesh=mesh,
        compiler_params=pltpu.CompilerParams(use_tc_tiling_on_sc=False),
        out_type=jax.ShapeDtypeStruct((3, 2, _NTEC, _NRC, _RCH, 64),
                                      jnp.float32),
        scratch_types=[
            pltpu.VMEM_SHARED((_NP + _TRASH, 64), jnp.float32),
            pltpu.VMEM_SHARED((_NP + _TRASH, 64), jnp.float32),
            pltpu.VMEM((_RCH, 64), jnp.float32),
            pltpu.VMEM((_RCH, 64), jnp.float32),
            pltpu.VMEM((_RCH, 64), jnp.float32),
            pltpu.VMEM((_RCH, 64), jnp.float32),
            pltpu.VMEM((_RCH, 64), jnp.float32),
            pltpu.VMEM((1, _ECH), jnp.int32),
            pltpu.VMEM((1, _ECH), jnp.int32),
            pltpu.VMEM((1, _ECH), jnp.int32),
            pltpu.VMEM((1, _ECH), jnp.int32),
            pltpu.VMEM((_RCH, 16), jnp.float32),
            pltpu.VMEM((8, 16), jnp.float32),
            pltpu.SemaphoreType.DMA,
            pltpu.SemaphoreType.DMA,
        ],
    )
    def kfn(u0_h, dsti_h, srci_h, rdeg_h, coefs_h, acc_h,
            xbuf, pbuf, stga, stgb, pchunk, xchunk, achunk,
            didxa, didxb, sidxa, sidxb, rdegc, cbuf, sema, semb):
        c = lax.axis_index("c")
        s = lax.axis_index("s")

        def zero_rows(buf, nrows):
            def zr(r, _):
                for col in range(4):
                    buf[r, pl.ds(col * 16, 16)] = jnp.zeros((16,), jnp.float32)
                return 0
            lax.fori_loop(0, nrows, zr, 0)

        zero_rows(pchunk, _RCH)

        @pl.when(s == 0)
        def _():
            pltpu.sync_copy(pchunk.at[pl.ds(0, _TRASH)],
                            xbuf.at[pl.ds(_NP, _TRASH)])
            pltpu.sync_copy(pchunk.at[pl.ds(0, _TRASH)],
                            pbuf.at[pl.ds(_NP, _TRASH)])

        def scale_rows(dst_buf, src_buf):
            def sr(r, _):
                rv = rdegc[r, pl.ds(0, 16)]
                for col in range(4):
                    dst_buf[r, pl.ds(col * 16, 16)] = (
                        src_buf[r, pl.ds(col * 16, 16)] * rv)
                return 0
            lax.fori_loop(0, _RCH, sr, 0)

        def init_chunk(j, _):
            row = s * _RPT + j * _RCH
            pltpu.sync_copy(rdeg_h.at[s, j], rdegc)
            pltpu.sync_copy(u0_h.at[c, s, j], achunk)
            pltpu.sync_copy(achunk, acc_h.at[0, c, s, j])
            scale_rows(xchunk, achunk)
            pltpu.sync_copy(xchunk, xbuf.at[pl.ds(row, _RCH)])
            pltpu.sync_copy(pchunk, pbuf.at[pl.ds(row, _RCH)])
            return 0
        lax.fori_loop(0, _NRC, init_chunk, 0)
        plsc.subcore_barrier()

        def load_idx(j, db, sb):
            pltpu.sync_copy(dsti_h.at[s, j], db)
            pltpu.sync_copy(srci_h.at[s, j], sb)

        def gather(db, stg, sem):
            return pltpu.make_async_copy(xbuf.at[db.at[0]], stg, sem)

        def scatter(stg, sb):
            pltpu.sync_copy(stg, pbuf.at[sb.at[0]], add=True)

        def phase_a():
            load_idx(0, didxa, sidxa)
            gather(didxa, stga, sema).start()

            def pair(jj, _):
                load_idx(2 * jj + 1, didxb, sidxb)
                gather(didxb, stgb, semb).start()
                gather(didxa, stga, sema).wait()
                scatter(stga, sidxa)

                @pl.when(jj < _NEC // 2 - 1)
                def _():
                    load_idx(2 * jj + 2, didxa, sidxa)
                    gather(didxa, stga, sema).start()
                gather(didxb, stgb, semb).wait()
                scatter(stgb, sidxb)
                return 0
            lax.fori_loop(0, _NEC // 2, pair, 0)

        def phase_b(k, t):
            pltpu.sync_copy(coefs_h.at[t], cbuf)
            crv = cbuf[0, pl.ds(0, 16)]
            civ = cbuf[1, pl.ds(0, 16)]

            def bchunk(j, _):
                row = s * _RPT + j * _RCH
                pltpu.sync_copy(rdeg_h.at[s, j], rdegc)
                pltpu.sync_copy(pbuf.at[pl.ds(row, _RCH)], pchunk)
                pltpu.sync_copy(acc_h.at[k, c, s, j], achunk)

                def brow(r, _):
                    rv = rdegc[r, pl.ds(0, 16)]
                    for h in range(2):
                        pr = pchunk[r, pl.ds(h * 16, 16)]
                        pi = pchunk[r, pl.ds(32 + h * 16, 16)]
                        ar = achunk[r, pl.ds(h * 16, 16)]
                        ai = achunk[r, pl.ds(32 + h * 16, 16)]
                        achunk[r, pl.ds(h * 16, 16)] = ar + crv * pr - civ * pi
                        achunk[r, pl.ds(32 + h * 16, 16)] = (
                            ai + crv * pi + civ * pr)
                        xchunk[r, pl.ds(h * 16, 16)] = pr * rv
                        xchunk[r, pl.ds(32 + h * 16, 16)] = pi * rv
                        pchunk[r, pl.ds(h * 16, 16)] = jnp.zeros(
                            (16,), jnp.float32)
                        pchunk[r, pl.ds(32 + h * 16, 16)] = jnp.zeros(
                            (16,), jnp.float32)
                    return 0
                lax.fori_loop(0, _RCH, brow, 0)
                pltpu.sync_copy(achunk, acc_h.at[k, c, s, j])
                pltpu.sync_copy(xchunk, xbuf.at[pl.ds(row, _RCH)])
                pltpu.sync_copy(pchunk, pbuf.at[pl.ds(row, _RCH)])
                return 0
            lax.fori_loop(0, _NRC, bchunk, 0)

        def boundary(k):
            def dchunk(j, _):
                row = s * _RPT + j * _RCH
                pltpu.sync_copy(rdeg_h.at[s, j], rdegc)
                pltpu.sync_copy(acc_h.at[k, c, s, j], achunk)
                scale_rows(xchunk, achunk)
                pltpu.sync_copy(xchunk, xbuf.at[pl.ds(row, _RCH)])
                if k < 2:
                    pltpu.sync_copy(achunk, acc_h.at[k + 1, c, s, j])
                return 0
            lax.fori_loop(0, _NRC, dchunk, 0)

        for k in range(3):
            def step(t, _):
                phase_a()
                plsc.subcore_barrier()
                phase_b(k, t)
                plsc.subcore_barrier()
                return 0
            lax.fori_loop(0, _T - 1, step, 0)
            boundary(k)
            plsc.subcore_barrier()

    return kfn(u0, dsti, srci, rdeg16, coefs)


def _tc_head(x, R, rs, W0, b0, W1, b1, W2, b2, Wh, bh, Wo, bo):
    nch = 20
    rows = _NP // nch

    def body(x_ref, r_ref, rs_ref, w0_ref, b0_ref, w1_ref, b1_ref, w2_ref,
             b2_ref, wh_ref, bhh_ref, wo_ref, bo_ref, out_ref, a_ref, q_ref):
        i = pl.program_id(0)

        @pl.when(i == 0)
        def _():
            a_ref[...] = jnp.zeros_like(a_ref)
            q_ref[...] = jnp.zeros_like(q_ref)

        rb = r_ref[...]
        xb = x_ref[...]
        rsb = rs_ref[...]
        r3 = jnp.concatenate([rb[2, 0, :, 0:32], rb[2, 1, :, 0:32]],
                             axis=1) * rsb
        a_ref[...] += lax.dot_general(
            r3, xb, (((0,), (0,)), ((), ())),
            precision=lax.Precision.HIGHEST,
            preferred_element_type=jnp.float32)
        for kk in range(3):
            rk = jnp.concatenate([rb[kk, 0, :, 0:32], rb[kk, 1, :, 0:32]],
                                 axis=1) * rsb
            q_ref[kk, 0:64] += jnp.sum(rk, axis=0)

        @pl.when(i == nch - 1)
        def _():
            mm = functools.partial(
                jnp.dot, precision=lax.Precision.HIGHEST,
                preferred_element_type=jnp.float32)
            a = a_ref[...]
            q3 = q_ref[2, 0:64][:, None]
            q2 = q_ref[1, 0:64][:, None]
            q1 = q_ref[0, 0:64][:, None]
            p = mm(a, w0_ref[...]) + q3 * b0_ref[...]
            p = mm(p, w1_ref[...]) + q2 * b1_ref[...]
            p = mm(p, w2_ref[...]) + q1 * b2_ref[...]
            h = jnp.maximum(mm(p, wh_ref[...]) + bhh_ref[...], 0.0)
            out_ref[...] = mm(h, wo_ref[...]) + bo_ref[...]

    full = lambda shp: pl.BlockSpec(shp, lambda i: tuple(0 for _ in shp))
    return pl.pallas_call(
        body,
        grid=(nch,),
        in_specs=[
            pl.BlockSpec((rows, _D), lambda i: (i, 0)),
            pl.BlockSpec((3, 2, rows, 64), lambda i: (0, 0, i, 0)),
            pl.BlockSpec((rows, 1), lambda i: (i, 0)),
            full((_D, _H)), full((1, _H)), full((_H, _H)), full((1, _H)),
            full((_H, _H)), full((1, _H)), full((_H, _H)), full((1, _H)),
            full((_H, _OUT)), full((1, _OUT)),
        ],
        out_specs=pl.BlockSpec((_G, _OUT), lambda i: (0, 0)),
        out_shape=jax.ShapeDtypeStruct((_G, _OUT), jnp.float32),
        scratch_shapes=[
            pltpu.VMEM((_G, _D), jnp.float32),
            pltpu.VMEM((8, 128), jnp.float32),
        ],
    )(x, R, rs, W0, b0, W1, b1, W2, b2, Wh, bh, Wo, bo)


def kernel(x, edge_index, batch, W0, b0, W1, b1, W2, b2, Wh, bh, Wo, bo):
    src = edge_index[0].astype(jnp.int32)
    dst = edge_index[1].astype(jnp.int32)
    deg = jnp.bincount(dst, length=_N).astype(jnp.float32) + 1.0
    degp = jnp.concatenate([deg, jnp.ones((_NP - _N,), jnp.float32)])
    rdeg = 1.0 / degp
    rs = jax.lax.rsqrt(degp)
    counts = jnp.clip(jnp.bincount(batch, length=_G).astype(jnp.float32),
                      1.0, None)

    pt = (jax.nn.one_hot(batch, _G, dtype=jnp.float32) / counts[None, :])
    u0r = jnp.sqrt(deg)[:, None] * pt
    u0 = jnp.zeros((2, _NP, 64), jnp.float32)
    u0 = u0.at[0, :_N, 0:32].set(u0r[:, 0:32])
    u0 = u0.at[1, :_N, 0:32].set(u0r[:, 32:64])
    u0 = u0.reshape(2, _NTEC, _NRC, _RCH, 64)

    pad = _NEC * _ECH - _EPT
    def chunked(a):
        a = a.reshape(_NTEC, _EPT)
        a = jnp.concatenate(
            [a, jnp.full((_NTEC, pad), _NP, jnp.int32)], axis=1)
        return a.reshape(_NTEC, _NEC, 1, _ECH)
    dsti = chunked(dst)
    srci = chunked(src)

    rdeg16 = jnp.broadcast_to(rdeg[:, None], (_NP, 16)).reshape(
        _NTEC, _NRC, _RCH, 16)

    import numpy as _np
    cr, ci, f = [], [], 1.0
    for t in range(1, _T):
        f *= t
        cr.append([1.0 / f, 0.0, -1.0 / f, 0.0][t % 4])
        ci.append([0.0, 1.0 / f, 0.0, -1.0 / f][t % 4])
    ctab = _np.zeros((_T - 1, 8, 16), _np.float32)
    ctab[:, 0, :] = _np.array(cr, _np.float32)[:, None]
    ctab[:, 1, :] = _np.array(ci, _np.float32)[:, None]
    coefs = jnp.asarray(ctab)

    U = _sc_taylor(u0, dsti, srci, rdeg16, coefs)
    R = U.transpose(0, 1, 2, 3, 4, 5).reshape(3, 2, _NP, 64)

    xp = jnp.zeros((_NP, _D), jnp.float32).at[:_N].set(x)
    return _tc_head(xp, R, rs[:, None], W0, b0.reshape(1, _H),
                    W1, b1.reshape(1, _H), W2, b2.reshape(1, _H),
                    Wh, bh.reshape(1, _H), Wo, bo.reshape(1, _OUT))


# speedup vs baseline: 12.1703x; 1.2323x over previous
import functools

import jax
import jax.numpy as jnp
from jax import lax
from jax.experimental import pallas as pl
from jax.experimental.pallas import tpu as pltpu
from jax.experimental.pallas import tpu_sc as plsc

_N = 10000
_E = 160000
_D = 128
_H = 128
_OUT = 40
_G = 64
_T = 20

_NTEC = 16
_NP = 10240
_RPT = _NP // _NTEC
_RCH = 128
_NRC = _RPT // _RCH
_ECH = 128
_EPT = _E // _NTEC
_NEC = -(-_EPT // _ECH)
_NEC = ((_NEC + 1) // 2) * 2
_TRASH = 8


def _sc_taylor(u0, idx, rdeg16, coefs):
    mesh = plsc.VectorSubcoreMesh(core_axis_name="c", subcore_axis_name="s")

    @functools.partial(
        pl.kernel,
        mesh=mesh,
        compiler_params=pltpu.CompilerParams(use_tc_tiling_on_sc=False),
        out_type=jax.ShapeDtypeStruct((3, 2, _NTEC, _NRC, _RCH, 64),
                                      jnp.float32),
        scratch_types=[
            pltpu.VMEM_SHARED((_NP + _TRASH, 64), jnp.float32),
            pltpu.VMEM_SHARED((_NP + _TRASH, 64), jnp.float32),
            pltpu.VMEM((_RCH, 64), jnp.float32),
            pltpu.VMEM((_RCH, 64), jnp.float32),
            pltpu.VMEM((_RCH, 64), jnp.float32),
            pltpu.VMEM((_RCH, 64), jnp.float32),
            pltpu.VMEM((_RCH, 64), jnp.float32),
            pltpu.VMEM((2, _ECH), jnp.int32),
            pltpu.VMEM((2, _ECH), jnp.int32),
            pltpu.VMEM((2, _ECH), jnp.int32),
            pltpu.VMEM((2, _ECH), jnp.int32),
            pltpu.VMEM((_RCH, 16), jnp.float32),
            pltpu.VMEM((8, 16), jnp.float32),
            pltpu.SemaphoreType.DMA,
            pltpu.SemaphoreType.DMA,
            pltpu.SemaphoreType.DMA,
            pltpu.SemaphoreType.DMA,
        ],
    )
    def kfn(u0_h, idx_h, rdeg_h, coefs_h, acc_h,
            xbuf, pbuf, stga, stgb, pchunk, xchunk, achunk,
            ia0, ib0, ia1, ib1, rdegc, cbuf, sga, sgb, ssa, ssb):
        c = lax.axis_index("c")
        s = lax.axis_index("s")

        def zero_rows(buf, nrows):
            def zr(r, _):
                for col in range(4):
                    buf[r, pl.ds(col * 16, 16)] = jnp.zeros((16,), jnp.float32)
                return 0
            lax.fori_loop(0, nrows, zr, 0)

        zero_rows(pchunk, _RCH)

        @pl.when(s == 0)
        def _():
            pltpu.sync_copy(pchunk.at[pl.ds(0, _TRASH)],
                            xbuf.at[pl.ds(_NP, _TRASH)])
            pltpu.sync_copy(pchunk.at[pl.ds(0, _TRASH)],
                            pbuf.at[pl.ds(_NP, _TRASH)])

        def scale_rows(dst_buf, src_buf):
            def sr(r, _):
                rv = rdegc[r, pl.ds(0, 16)]
                for col in range(4):
                    dst_buf[r, pl.ds(col * 16, 16)] = (
                        src_buf[r, pl.ds(col * 16, 16)] * rv)
                return 0
            lax.fori_loop(0, _RCH, sr, 0)

        def init_chunk(j, _):
            row = s * _RPT + j * _RCH
            pltpu.sync_copy(rdeg_h.at[s, j], rdegc)
            pltpu.sync_copy(u0_h.at[c, s, j], achunk)
            pltpu.sync_copy(achunk, acc_h.at[0, c, s, j])
            scale_rows(xchunk, achunk)
            pltpu.sync_copy(xchunk, xbuf.at[pl.ds(row, _RCH)])
            pltpu.sync_copy(pchunk, pbuf.at[pl.ds(row, _RCH)])
            return 0
        lax.fori_loop(0, _NRC, init_chunk, 0)
        plsc.subcore_barrier()

        def load_idx(j, ib):
            pltpu.sync_copy(idx_h.at[s, j], ib)

        def gather(ib, stg, sem):
            return pltpu.make_async_copy(xbuf.at[ib.at[0]], stg, sem)

        def scatter(stg, ib, sem):
            return pltpu.async_copy(stg, pbuf.at[ib.at[1]], sem, add=True)

        def phase_a():
            load_idx(0, ia0)
            load_idx(1, ib0)
            load_idx(2, ia1)
            load_idx(3, ib1)
            gather(ia0, stga, sga).start()
            gather(ib0, stgb, sgb).start()

            nq = _NEC // 4

            def quad(q, _):
                j = 4 * q
                gather(ia0, stga, sga).wait()
                sa = scatter(stga, ia0, ssa)
                gather(ib0, stgb, sgb).wait()
                sb = scatter(stgb, ib0, ssb)
                sa.wait()
                gather(ia1, stga, sga).start()
                sb.wait()
                gather(ib1, stgb, sgb).start()

                @pl.when(q < nq - 1)
                def _():
                    load_idx(j + 4, ia0)
                    load_idx(j + 5, ib0)
                gather(ia1, stga, sga).wait()
                sa2 = scatter(stga, ia1, ssa)
                gather(ib1, stgb, sgb).wait()
                sb2 = scatter(stgb, ib1, ssb)
                sa2.wait()
                sb2.wait()

                @pl.when(q < nq - 1)
                def _():
                    gather(ia0, stga, sga).start()
                    gather(ib0, stgb, sgb).start()
                    load_idx(j + 6, ia1)
                    load_idx(j + 7, ib1)
                return 0
            lax.fori_loop(0, nq, quad, 0)

        def phase_b(k, t):
            pltpu.sync_copy(coefs_h.at[t], cbuf)
            crv = cbuf[0, pl.ds(0, 16)]
            civ = cbuf[1, pl.ds(0, 16)]

            def bchunk(j, _):
                row = s * _RPT + j * _RCH
                pltpu.sync_copy(rdeg_h.at[s, j], rdegc)
                pltpu.sync_copy(pbuf.at[pl.ds(row, _RCH)], pchunk)
                pltpu.sync_copy(acc_h.at[k, c, s, j], achunk)

                def brow(r, _):
                    rv = rdegc[r, pl.ds(0, 16)]
                    for h in range(2):
                        pr = pchunk[r, pl.ds(h * 16, 16)]
                        pi = pchunk[r, pl.ds(32 + h * 16, 16)]
                        ar = achunk[r, pl.ds(h * 16, 16)]
                        ai = achunk[r, pl.ds(32 + h * 16, 16)]
                        achunk[r, pl.ds(h * 16, 16)] = ar + crv * pr - civ * pi
                        achunk[r, pl.ds(32 + h * 16, 16)] = (
                            ai + crv * pi + civ * pr)
                        xchunk[r, pl.ds(h * 16, 16)] = pr * rv
                        xchunk[r, pl.ds(32 + h * 16, 16)] = pi * rv
                        pchunk[r, pl.ds(h * 16, 16)] = jnp.zeros(
                            (16,), jnp.float32)
                        pchunk[r, pl.ds(32 + h * 16, 16)] = jnp.zeros(
                            (16,), jnp.float32)
                    return 0
                lax.fori_loop(0, _RCH, brow, 0)
                pltpu.sync_copy(achunk, acc_h.at[k, c, s, j])
                pltpu.sync_copy(xchunk, xbuf.at[pl.ds(row, _RCH)])
                pltpu.sync_copy(pchunk, pbuf.at[pl.ds(row, _RCH)])
                return 0
            lax.fori_loop(0, _NRC, bchunk, 0)

        def boundary(k):
            def dchunk(j, _):
                row = s * _RPT + j * _RCH
                pltpu.sync_copy(rdeg_h.at[s, j], rdegc)
                pltpu.sync_copy(acc_h.at[k, c, s, j], achunk)
                scale_rows(xchunk, achunk)
                pltpu.sync_copy(xchunk, xbuf.at[pl.ds(row, _RCH)])
                if k < 2:
                    pltpu.sync_copy(achunk, acc_h.at[k + 1, c, s, j])
                return 0
            lax.fori_loop(0, _NRC, dchunk, 0)

        for k in range(3):
            def step(t, _):
                phase_a()
                plsc.subcore_barrier()
                phase_b(k, t)
                plsc.subcore_barrier()
                return 0
            lax.fori_loop(0, _T - 1, step, 0)
            boundary(k)
            plsc.subcore_barrier()

    return kfn(u0, idx, rdeg16, coefs)


def _tc_head(x, R, rs, W0, b0, W1, b1, W2, b2, Wh, bh, Wo, bo):
    nch = 20
    rows = _NP // nch

    def body(x_ref, r_ref, rs_ref, w0_ref, b0_ref, w1_ref, b1_ref, w2_ref,
             b2_ref, wh_ref, bhh_ref, wo_ref, bo_ref, out_ref, a_ref, q_ref):
        i = pl.program_id(0)

        @pl.when(i == 0)
        def _():
            a_ref[...] = jnp.zeros_like(a_ref)
            q_ref[...] = jnp.zeros_like(q_ref)

        rb = r_ref[...]
        xb = x_ref[...]
        rsb = rs_ref[...]
        r3 = jnp.concatenate([rb[2, 0, :, 0:32], rb[2, 1, :, 0:32]],
                             axis=1) * rsb
        a_ref[...] += lax.dot_general(
            r3, xb, (((0,), (0,)), ((), ())),
            precision=lax.Precision.HIGHEST,
            preferred_element_type=jnp.float32)
        for kk in range(3):
            rk = jnp.concatenate([rb[kk, 0, :, 0:32], rb[kk, 1, :, 0:32]],
                                 axis=1) * rsb
            q_ref[kk, 0:64] += jnp.sum(rk, axis=0)

        @pl.when(i == nch - 1)
        def _():
            mm = functools.partial(
                jnp.dot, precision=lax.Precision.HIGHEST,
                preferred_element_type=jnp.float32)
            a = a_ref[...]
            q3 = q_ref[2, 0:64][:, None]
            q2 = q_ref[1, 0:64][:, None]
            q1 = q_ref[0, 0:64][:, None]
            p = mm(a, w0_ref[...]) + q3 * b0_ref[...]
            p = mm(p, w1_ref[...]) + q2 * b1_ref[...]
            p = mm(p, w2_ref[...]) + q1 * b2_ref[...]
            h = jnp.maximum(mm(p, wh_ref[...]) + bhh_ref[...], 0.0)
            out_ref[...] = mm(h, wo_ref[...]) + bo_ref[...]

    full = lambda shp: pl.BlockSpec(shp, lambda i: tuple(0 for _ in shp))
    return pl.pallas_call(
        body,
        grid=(nch,),
        in_specs=[
            pl.BlockSpec((rows, _D), lambda i: (i, 0)),
            pl.BlockSpec((3, 2, rows, 64), lambda i: (0, 0, i, 0)),
            pl.BlockSpec((rows, 1), lambda i: (i, 0)),
            full((_D, _H)), full((1, _H)), full((_H, _H)), full((1, _H)),
            full((_H, _H)), full((1, _H)), full((_H, _H)), full((1, _H)),
            full((_H, _OUT)), full((1, _OUT)),
        ],
        out_specs=pl.BlockSpec((_G, _OUT), lambda i: (0, 0)),
        out_shape=jax.ShapeDtypeStruct((_G, _OUT), jnp.float32),
        scratch_shapes=[
            pltpu.VMEM((_G, _D), jnp.float32),
            pltpu.VMEM((8, 128), jnp.float32),
        ],
    )(x, R, rs, W0, b0, W1, b1, W2, b2, Wh, bh, Wo, bo)


def kernel(x, edge_index, batch, W0, b0, W1, b1, W2, b2, Wh, bh, Wo, bo):
    src = edge_index[0].astype(jnp.int32)
    dst = edge_index[1].astype(jnp.int32)
    deg = jnp.bincount(dst, length=_N).astype(jnp.float32) + 1.0
    degp = jnp.concatenate([deg, jnp.ones((_NP - _N,), jnp.float32)])
    rdeg = 1.0 / degp
    rs = jax.lax.rsqrt(degp)
    counts = jnp.clip(jnp.bincount(batch, length=_G).astype(jnp.float32),
                      1.0, None)

    pt = (jax.nn.one_hot(batch, _G, dtype=jnp.float32) / counts[None, :])
    u0r = jnp.sqrt(deg)[:, None] * pt
    u0 = jnp.zeros((2, _NP, 64), jnp.float32)
    u0 = u0.at[0, :_N, 0:32].set(u0r[:, 0:32])
    u0 = u0.at[1, :_N, 0:32].set(u0r[:, 32:64])
    u0 = u0.reshape(2, _NTEC, _NRC, _RCH, 64)

    pad = _NEC * _ECH - _EPT
    def chunked(a):
        a = a.reshape(_NTEC, _EPT)
        a = jnp.concatenate(
            [a, jnp.full((_NTEC, pad), _NP, jnp.int32)], axis=1)
        return a.reshape(_NTEC, _NEC, 1, _ECH)
    idx = jnp.concatenate([chunked(dst), chunked(src)], axis=2)

    rdeg16 = jnp.broadcast_to(rdeg[:, None], (_NP, 16)).reshape(
        _NTEC, _NRC, _RCH, 16)

    import numpy as _np
    cr, ci, f = [], [], 1.0
    for t in range(1, _T):
        f *= t
        cr.append([1.0 / f, 0.0, -1.0 / f, 0.0][t % 4])
        ci.append([0.0, 1.0 / f, 0.0, -1.0 / f][t % 4])
    ctab = _np.zeros((_T - 1, 8, 16), _np.float32)
    ctab[:, 0, :] = _np.array(cr, _np.float32)[:, None]
    ctab[:, 1, :] = _np.array(ci, _np.float32)[:, None]
    coefs = jnp.asarray(ctab)

    U = _sc_taylor(u0, idx, rdeg16, coefs)
    R = U.transpose(0, 1, 2, 3, 4, 5).reshape(3, 2, _NP, 64)

    xp = jnp.zeros((_NP, _D), jnp.float32).at[:_N].set(x)
    return _tc_head(xp, R, rs[:, None], W0, b0.reshape(1, _H),
                    W1, b1.reshape(1, _H), W2, b2.reshape(1, _H),
                    Wh, bh.reshape(1, _H), Wo, bo.reshape(1, _OUT))


# speedup vs baseline: 12.1862x; 1.0013x over previous
import functools

import jax
import jax.numpy as jnp
from jax import lax
from jax.experimental import pallas as pl
from jax.experimental.pallas import tpu as pltpu
from jax.experimental.pallas import tpu_sc as plsc

_N = 10000
_E = 160000
_D = 128
_H = 128
_OUT = 40
_G = 64
_T = 20

_NTEC = 16
_NP = 10240
_RPT = _NP // _NTEC
_RCH = 128
_NRC = _RPT // _RCH
_ECH = 128
_EPT = _E // _NTEC
_NEC = -(-_EPT // _ECH)
_NEC = ((_NEC + 1) // 2) * 2
_TRASH = 8


def _sc_taylor(u0, idx, rdeg16, coefs):
    mesh = plsc.VectorSubcoreMesh(core_axis_name="c", subcore_axis_name="s")

    @functools.partial(
        pl.kernel,
        mesh=mesh,
        compiler_params=pltpu.CompilerParams(use_tc_tiling_on_sc=False),
        out_type=jax.ShapeDtypeStruct((3, 2, _NTEC, _NRC, _RCH, 64),
                                      jnp.float32),
        scratch_types=[
            pltpu.VMEM_SHARED((_NP + _TRASH, 64), jnp.float32),
            pltpu.VMEM_SHARED((_NP + _TRASH, 64), jnp.float32),
            pltpu.VMEM((_RCH, 64), jnp.float32),
            pltpu.VMEM((_RCH, 64), jnp.float32),
            pltpu.VMEM((_RCH, 64), jnp.float32),
            pltpu.VMEM((_RCH, 64), jnp.float32),
            pltpu.VMEM((_RCH, 64), jnp.float32),
            pltpu.VMEM((2, _ECH), jnp.int32),
            pltpu.VMEM((2, _ECH), jnp.int32),
            pltpu.VMEM((2, _ECH), jnp.int32),
            pltpu.VMEM((2, _ECH), jnp.int32),
            pltpu.VMEM((_RCH, 16), jnp.float32),
            pltpu.VMEM((_RCH, 16), jnp.float32),
            pltpu.VMEM((8, 16), jnp.float32),
            pltpu.SemaphoreType.DMA,
            pltpu.SemaphoreType.DMA,
            pltpu.SemaphoreType.DMA,
            pltpu.SemaphoreType.DMA,
        ],
    )
    def kfn(u0_h, idx_h, rdeg_h, coefs_h, acc_h,
            xbuf, pbuf, stga, stgb, pchunk, xchunk, achunk,
            ia0, ib0, ia1, ib1, rdegc, rdegc2, cbuf, sga, sgb, ssa, ssb):
        c = lax.axis_index("c")
        s = lax.axis_index("s")

        def zero_rows(buf, nrows):
            def zr(r, _):
                for col in range(4):
                    buf[r, pl.ds(col * 16, 16)] = jnp.zeros((16,), jnp.float32)
                return 0
            lax.fori_loop(0, nrows, zr, 0)

        zero_rows(pchunk, _RCH)

        @pl.when(s == 0)
        def _():
            pltpu.sync_copy(pchunk.at[pl.ds(0, _TRASH)],
                            xbuf.at[pl.ds(_NP, _TRASH)])
            pltpu.sync_copy(pchunk.at[pl.ds(0, _TRASH)],
                            pbuf.at[pl.ds(_NP, _TRASH)])

        def scale_rows(dst_buf, src_buf):
            def sr(r, _):
                rv = rdegc[r, pl.ds(0, 16)]
                for col in range(4):
                    dst_buf[r, pl.ds(col * 16, 16)] = (
                        src_buf[r, pl.ds(col * 16, 16)] * rv)
                return 0
            lax.fori_loop(0, _RCH, sr, 0)

        def init_chunk(j, _):
            row = s * _RPT + j * _RCH
            pltpu.sync_copy(rdeg_h.at[s, j], rdegc)
            pltpu.sync_copy(u0_h.at[c, s, j], achunk)
            pltpu.sync_copy(achunk, acc_h.at[0, c, s, j])
            scale_rows(xchunk, achunk)
            pltpu.sync_copy(xchunk, xbuf.at[pl.ds(row, _RCH)])
            pltpu.sync_copy(pchunk, pbuf.at[pl.ds(row, _RCH)])
            return 0
        lax.fori_loop(0, _NRC, init_chunk, 0)
        plsc.subcore_barrier()

        def load_idx(j, ib):
            pltpu.sync_copy(idx_h.at[s, j], ib)

        def gather(ib, stg, sem):
            return pltpu.make_async_copy(xbuf.at[ib.at[0]], stg, sem)

        def scatter(stg, ib, sem):
            return pltpu.async_copy(stg, pbuf.at[ib.at[1]], sem, add=True)

        def phase_a():
            load_idx(0, ia0)
            load_idx(1, ib0)
            load_idx(2, ia1)
            load_idx(3, ib1)
            gather(ia0, stga, sga).start()
            gather(ib0, stgb, sgb).start()

            nq = _NEC // 4

            def quad(q, _):
                j = 4 * q
                gather(ia0, stga, sga).wait()
                sa = scatter(stga, ia0, ssa)
                gather(ib0, stgb, sgb).wait()
                sb = scatter(stgb, ib0, ssb)
                sa.wait()
                gather(ia1, stga, sga).start()
                sb.wait()
                gather(ib1, stgb, sgb).start()

                @pl.when(q < nq - 1)
                def _():
                    load_idx(j + 4, ia0)
                    load_idx(j + 5, ib0)
                gather(ia1, stga, sga).wait()
                sa2 = scatter(stga, ia1, ssa)
                gather(ib1, stgb, sgb).wait()
                sb2 = scatter(stgb, ib1, ssb)
                sa2.wait()
                sb2.wait()

                @pl.when(q < nq - 1)
                def _():
                    gather(ia0, stga, sga).start()
                    gather(ib0, stgb, sgb).start()
                    load_idx(j + 6, ia1)
                    load_idx(j + 7, ib1)
                return 0
            lax.fori_loop(0, nq, quad, 0)

        def phase_b(k, t):
            pltpu.sync_copy(coefs_h.at[t], cbuf)
            crv = cbuf[0, pl.ds(0, 16)]
            civ = cbuf[1, pl.ds(0, 16)]
            pairs = [(pchunk, achunk, rdegc, sga), (stga, stgb, rdegc2, sgb)]

            def row_loop(pc, ac, rc):
                def brow(r, _):
                    rv = rc[r, pl.ds(0, 16)]
                    for h in range(2):
                        pr = pc[r, pl.ds(h * 16, 16)]
                        pi = pc[r, pl.ds(32 + h * 16, 16)]
                        ar = ac[r, pl.ds(h * 16, 16)]
                        ai = ac[r, pl.ds(32 + h * 16, 16)]
                        ac[r, pl.ds(h * 16, 16)] = ar + crv * pr - civ * pi
                        ac[r, pl.ds(32 + h * 16, 16)] = (
                            ai + crv * pi + civ * pr)
                        xchunk[r, pl.ds(h * 16, 16)] = pr * rv
                        xchunk[r, pl.ds(32 + h * 16, 16)] = pi * rv
                        pc[r, pl.ds(h * 16, 16)] = jnp.zeros(
                            (16,), jnp.float32)
                        pc[r, pl.ds(32 + h * 16, 16)] = jnp.zeros(
                            (16,), jnp.float32)
                    return 0
                lax.fori_loop(0, _RCH, brow, 0)

            for j in range(_NRC):
                pc, ac, rc, sem = pairs[j % 2]
                row = s * _RPT + j * _RCH
                pltpu.sync_copy(rdeg_h.at[s, j], rc)
                pltpu.sync_copy(pbuf.at[pl.ds(row, _RCH)], pc)
                pltpu.sync_copy(acc_h.at[k, c, s, j], ac)
                row_loop(pc, ac, rc)
                pltpu.sync_copy(ac, acc_h.at[k, c, s, j])
                pltpu.sync_copy(xchunk, xbuf.at[pl.ds(row, _RCH)])
                pltpu.sync_copy(pc, pbuf.at[pl.ds(row, _RCH)])

        def boundary(k):
            def dchunk(j, _):
                row = s * _RPT + j * _RCH
                pltpu.sync_copy(rdeg_h.at[s, j], rdegc)
                pltpu.sync_copy(acc_h.at[k, c, s, j], achunk)
                scale_rows(xchunk, achunk)
                pltpu.sync_copy(xchunk, xbuf.at[pl.ds(row, _RCH)])
                if k < 2:
                    pltpu.sync_copy(achunk, acc_h.at[k + 1, c, s, j])
                return 0
            lax.fori_loop(0, _NRC, dchunk, 0)

        for k in range(3):
            def step(t, _):
                phase_a()
                plsc.subcore_barrier()
                phase_b(k, t)
                plsc.subcore_barrier()
                return 0
            lax.fori_loop(0, _T - 1, step, 0)
            boundary(k)
            plsc.subcore_barrier()

    return kfn(u0, idx, rdeg16, coefs)


def _tc_head(x, R, rs, W0, b0, W1, b1, W2, b2, Wh, bh, Wo, bo):
    nch = 20
    rows = _NP // nch

    def body(x_ref, r_ref, rs_ref, w0_ref, b0_ref, w1_ref, b1_ref, w2_ref,
             b2_ref, wh_ref, bhh_ref, wo_ref, bo_ref, out_ref, a_ref, q_ref):
        i = pl.program_id(0)

        @pl.when(i == 0)
        def _():
            a_ref[...] = jnp.zeros_like(a_ref)
            q_ref[...] = jnp.zeros_like(q_ref)

        rb = r_ref[...]
        xb = x_ref[...]
        rsb = rs_ref[...]
        r3 = jnp.concatenate([rb[2, 0, :, 0:32], rb[2, 1, :, 0:32]],
                             axis=1) * rsb
        a_ref[...] += lax.dot_general(
            r3, xb, (((0,), (0,)), ((), ())),
            precision=lax.Precision.HIGHEST,
            preferred_element_type=jnp.float32)
        for kk in range(3):
            rk = jnp.concatenate([rb[kk, 0, :, 0:32], rb[kk, 1, :, 0:32]],
                                 axis=1) * rsb
            q_ref[kk, 0:64] += jnp.sum(rk, axis=0)

        @pl.when(i == nch - 1)
        def _():
            mm = functools.partial(
                jnp.dot, precision=lax.Precision.HIGHEST,
                preferred_element_type=jnp.float32)
            a = a_ref[...]
            q3 = q_ref[2, 0:64][:, None]
            q2 = q_ref[1, 0:64][:, None]
            q1 = q_ref[0, 0:64][:, None]
            p = mm(a, w0_ref[...]) + q3 * b0_ref[...]
            p = mm(p, w1_ref[...]) + q2 * b1_ref[...]
            p = mm(p, w2_ref[...]) + q1 * b2_ref[...]
            h = jnp.maximum(mm(p, wh_ref[...]) + bhh_ref[...], 0.0)
            out_ref[...] = mm(h, wo_ref[...]) + bo_ref[...]

    full = lambda shp: pl.BlockSpec(shp, lambda i: tuple(0 for _ in shp))
    return pl.pallas_call(
        body,
        grid=(nch,),
        in_specs=[
            pl.BlockSpec((rows, _D), lambda i: (i, 0)),
            pl.BlockSpec((3, 2, rows, 64), lambda i: (0, 0, i, 0)),
            pl.BlockSpec((rows, 1), lambda i: (i, 0)),
            full((_D, _H)), full((1, _H)), full((_H, _H)), full((1, _H)),
            full((_H, _H)), full((1, _H)), full((_H, _H)), full((1, _H)),
            full((_H, _OUT)), full((1, _OUT)),
        ],
        out_specs=pl.BlockSpec((_G, _OUT), lambda i: (0, 0)),
        out_shape=jax.ShapeDtypeStruct((_G, _OUT), jnp.float32),
        scratch_shapes=[
            pltpu.VMEM((_G, _D), jnp.float32),
            pltpu.VMEM((8, 128), jnp.float32),
        ],
    )(x, R, rs, W0, b0, W1, b1, W2, b2, Wh, bh, Wo, bo)


def kernel(x, edge_index, batch, W0, b0, W1, b1, W2, b2, Wh, bh, Wo, bo):
    src = edge_index[0].astype(jnp.int32)
    dst = edge_index[1].astype(jnp.int32)
    deg = jnp.bincount(dst, length=_N).astype(jnp.float32) + 1.0
    degp = jnp.concatenate([deg, jnp.ones((_NP - _N,), jnp.float32)])
    rdeg = 1.0 / degp
    rs = jax.lax.rsqrt(degp)
    counts = jnp.clip(jnp.bincount(batch, length=_G).astype(jnp.float32),
                      1.0, None)

    pt = (jax.nn.one_hot(batch, _G, dtype=jnp.float32) / counts[None, :])
    u0r = jnp.sqrt(deg)[:, None] * pt
    u0 = jnp.zeros((2, _NP, 64), jnp.float32)
    u0 = u0.at[0, :_N, 0:32].set(u0r[:, 0:32])
    u0 = u0.at[1, :_N, 0:32].set(u0r[:, 32:64])
    u0 = u0.reshape(2, _NTEC, _NRC, _RCH, 64)

    pad = _NEC * _ECH - _EPT
    def chunked(a):
        a = a.reshape(_NTEC, _EPT)
        a = jnp.concatenate(
            [a, jnp.full((_NTEC, pad), _NP, jnp.int32)], axis=1)
        return a.reshape(_NTEC, _NEC, 1, _ECH)
    idx = jnp.concatenate([chunked(dst), chunked(src)], axis=2)

    rdeg16 = jnp.broadcast_to(rdeg[:, None], (_NP, 16)).reshape(
        _NTEC, _NRC, _RCH, 16)

    import numpy as _np
    cr, ci, f = [], [], 1.0
    for t in range(1, _T):
        f *= t
        cr.append([1.0 / f, 0.0, -1.0 / f, 0.0][t % 4])
        ci.append([0.0, 1.0 / f, 0.0, -1.0 / f][t % 4])
    ctab = _np.zeros((_T - 1, 8, 16), _np.float32)
    ctab[:, 0, :] = _np.array(cr, _np.float32)[:, None]
    ctab[:, 1, :] = _np.array(ci, _np.float32)[:, None]
    coefs = jnp.asarray(ctab)

    U = _sc_taylor(u0, idx, rdeg16, coefs)
    R = U.transpose(0, 1, 2, 3, 4, 5).reshape(3, 2, _NP, 64)

    xp = jnp.zeros((_NP, _D), jnp.float32).at[:_N].set(x)
    return _tc_head(xp, R, rs[:, None], W0, b0.reshape(1, _H),
                    W1, b1.reshape(1, _H), W2, b2.reshape(1, _H),
                    Wh, bh.reshape(1, _H), Wo, bo.reshape(1, _OUT))
